# baseline (speedup 1.0000x reference)
import jax
import jax.numpy as jnp
from jax.experimental import pallas as pl

N_NODES = 10000
WIDTH = 256
L = 8
G = 4
EPS = 1e-5

CIN_G = WIDTH // G
C1OUT = WIDTH * 8
C1OUT_G = C1OUT // G
C2IN = WIDTH * 4
C2IN_G = C2IN // G

BN = 500
R = BN * L
GRID = N_NODES // BN


def _block(xr_ref, w1_ref, w2_ref, b1_ref, b2_ref, sel_ref, o_ref):
    xb = xr_ref[...]
    sel = sel_ref[...]

    cs = jnp.dot(xb, sel, preferred_element_type=jnp.float32)
    ss = jnp.dot(xb * xb, sel, preferred_element_type=jnp.float32)
    both = jnp.concatenate([cs, ss], axis=1)
    b3 = jnp.sum(both.reshape(BN, L, 2 * G), axis=1, keepdims=True)
    bn = jnp.broadcast_to(b3, (BN, L, 2 * G)).reshape(R, 2 * G)
    cnt = float(CIN_G * L)
    mean = bn[:, :G] * (1.0 / cnt)
    var = bn[:, G:] * (1.0 / cnt) - mean * mean
    inv = jax.lax.rsqrt(var + EPS)
    selT = jnp.transpose(sel)
    scale = jnp.dot(inv, selT, preferred_element_type=jnp.float32)
    shift = jnp.dot(mean * inv, selT,
                    preferred_element_type=jnp.float32)
    xn_full = xb * scale - shift

    ys = []
    for g in range(G):
        xn = xn_full[:, g * CIN_G:(g + 1) * CIN_G]
        y = jnp.dot(xn, w1_ref[g], preferred_element_type=jnp.float32)
        ys.append(y + b1_ref[:, g * C1OUT_G:(g + 1) * C1OUT_G])

    h01 = jnp.maximum(ys[0], 0.0) * ys[2]
    h23 = jnp.maximum(ys[1], 0.0) * ys[3]

    halves = (h01[:, :C2IN_G], h01[:, C2IN_G:],
              h23[:, :C2IN_G], h23[:, C2IN_G:])
    for g in range(G):
        og = jnp.dot(halves[g], w2_ref[g],
                     preferred_element_type=jnp.float32)
        o_ref[:, g * CIN_G:(g + 1) * CIN_G] = (
            og + b2_ref[:, g * CIN_G:(g + 1) * CIN_G])


def kernel(x, edge_index, edge_attr, W1, b1, W2, b2):
    del edge_index, edge_attr
    xr = jnp.transpose(x, (0, 2, 1)).reshape(N_NODES * L, WIDTH)
    w1t = jnp.swapaxes(W1.reshape(G, C1OUT_G, CIN_G), 1, 2)
    w2t = jnp.swapaxes(W2.reshape(G, CIN_G, C2IN_G), 1, 2)
    b1r = b1.reshape(1, C1OUT)
    b2r = b2.reshape(1, WIDTH)
    sel = jnp.repeat(jnp.eye(G, dtype=jnp.float32), CIN_G, axis=0)

    out = pl.pallas_call(
        _block,
        grid=(GRID,),
        in_specs=[
            pl.BlockSpec((R, WIDTH), lambda i: (i, 0)),
            pl.BlockSpec((G, CIN_G, C1OUT_G), lambda i: (0, 0, 0)),
            pl.BlockSpec((G, C2IN_G, CIN_G), lambda i: (0, 0, 0)),
            pl.BlockSpec((1, C1OUT), lambda i: (0, 0)),
            pl.BlockSpec((1, WIDTH), lambda i: (0, 0)),
            pl.BlockSpec((WIDTH, G), lambda i: (0, 0)),
        ],
        out_specs=pl.BlockSpec((R, WIDTH), lambda i: (i, 0)),
        out_shape=jax.ShapeDtypeStruct((N_NODES * L, WIDTH), jnp.float32),
    )(xr, w1t, w2t, b1r, b2r, sel)

    return jnp.transpose(out.reshape(N_NODES, L, WIDTH), (0, 2, 1))


# speedup vs baseline: 1.0022x; 1.0022x over previous
import jax
import jax.numpy as jnp
from jax.experimental import pallas as pl

N_NODES = 10000
WIDTH = 256
L = 8
G = 4
EPS = 1e-5

CIN_G = WIDTH // G
C1OUT = WIDTH * 8
C1OUT_G = C1OUT // G
C2IN = WIDTH * 4
C2IN_G = C2IN // G

BN = 625
R = BN * L
GRID = N_NODES // BN


def _block(xr_ref, w1_ref, w2_ref, b1_ref, b2_ref, sel_ref, o_ref):
    xb = xr_ref[...]
    sel = sel_ref[...]

    cs = jnp.dot(xb, sel, preferred_element_type=jnp.float32)
    ss = jnp.dot(xb * xb, sel, preferred_element_type=jnp.float32)
    both = jnp.concatenate([cs, ss], axis=1)
    b3 = jnp.sum(both.reshape(BN, L, 2 * G), axis=1, keepdims=True)
    bn = jnp.broadcast_to(b3, (BN, L, 2 * G)).reshape(R, 2 * G)
    cnt = float(CIN_G * L)
    mean = bn[:, :G] * (1.0 / cnt)
    var = bn[:, G:] * (1.0 / cnt) - mean * mean
    inv = jax.lax.rsqrt(var + EPS)
    selT = jnp.transpose(sel)
    scale = jnp.dot(inv, selT, preferred_element_type=jnp.float32)
    shift = jnp.dot(mean * inv, selT,
                    preferred_element_type=jnp.float32)
    xn_full = xb * scale - shift

    ys = []
    for g in range(G):
        xn = xn_full[:, g * CIN_G:(g + 1) * CIN_G]
        y = jnp.dot(xn, w1_ref[g], preferred_element_type=jnp.float32)
        ys.append(y + b1_ref[:, g * C1OUT_G:(g + 1) * C1OUT_G])

    h01 = jnp.maximum(ys[0], 0.0) * ys[2]
    h23 = jnp.maximum(ys[1], 0.0) * ys[3]

    halves = (h01[:, :C2IN_G], h01[:, C2IN_G:],
              h23[:, :C2IN_G], h23[:, C2IN_G:])
    for g in range(G):
        og = jnp.dot(halves[g], w2_ref[g],
                     preferred_element_type=jnp.float32)
        o_ref[:, g * CIN_G:(g + 1) * CIN_G] = (
            og + b2_ref[:, g * CIN_G:(g + 1) * CIN_G])


def kernel(x, edge_index, edge_attr, W1, b1, W2, b2):
    del edge_index, edge_attr
    xr = jnp.transpose(x, (0, 2, 1)).reshape(N_NODES * L, WIDTH)
    w1t = jnp.swapaxes(W1.reshape(G, C1OUT_G, CIN_G), 1, 2)
    w2t = jnp.swapaxes(W2.reshape(G, CIN_G, C2IN_G), 1, 2)
    b1r = b1.reshape(1, C1OUT)
    b2r = b2.reshape(1, WIDTH)
    sel = jnp.repeat(jnp.eye(G, dtype=jnp.float32), CIN_G, axis=0)

    out = pl.pallas_call(
        _block,
        grid=(GRID,),
        in_specs=[
            pl.BlockSpec((R, WIDTH), lambda i: (i, 0)),
            pl.BlockSpec((G, CIN_G, C1OUT_G), lambda i: (0, 0, 0)),
            pl.BlockSpec((G, C2IN_G, CIN_G), lambda i: (0, 0, 0)),
            pl.BlockSpec((1, C1OUT), lambda i: (0, 0)),
            pl.BlockSpec((1, WIDTH), lambda i: (0, 0)),
            pl.BlockSpec((WIDTH, G), lambda i: (0, 0)),
        ],
        out_specs=pl.BlockSpec((R, WIDTH), lambda i: (i, 0)),
        out_shape=jax.ShapeDtypeStruct((N_NODES * L, WIDTH), jnp.float32),
    )(xr, w1t, w2t, b1r, b2r, sel)

    return jnp.transpose(out.reshape(N_NODES, L, WIDTH), (0, 2, 1))
